# baseline (speedup 1.0000x reference)
import jax
import jax.numpy as jnp
from jax.experimental import pallas as pl
from jax.experimental.pallas import tpu as pltpu

_EPS = 1e-6
_P1PAD = 7


def _pass1_kernel(x_ref, z_ref, h_ref, wcol_ref,
                  wpsi_ref, bpsi_ref, wphi_ref, bphi_ref,
                  wg1_ref, bg1_ref, wg2_ref, bg2_ref,
                  gate_ref, e1_ref, dv_ref):
    s = pl.program_id(1)
    half = wpsi_ref.shape[1]
    tn = x_ref.shape[0]
    px = jnp.dot(x_ref[:], wpsi_ref[:], preferred_element_type=jnp.float32) + bpsi_ref[:]
    pz = jnp.dot(z_ref[:], wphi_ref[:], preferred_element_type=jnp.float32) + bphi_ref[:]
    g1 = jax.nn.relu(
        jnp.dot(px, wg1_ref[0:half, :], preferred_element_type=jnp.float32)
        + jnp.dot(pz, wg1_ref[half:, :], preferred_element_type=jnp.float32)
        + bg1_ref[:])
    gate = jax.nn.sigmoid(
        jnp.dot(g1, wg2_ref[:], preferred_element_type=jnp.float32) + bg2_ref[:])
    fused = gate * pz + (1.0 - gate) * px
    gate_ref[:] = gate

    h = h_ref[:]
    dv_ref[:] = jnp.clip(
        jnp.dot(h, wcol_ref[:], preferred_element_type=jnp.float32)[:, 0:1],
        _EPS, None)

    @pl.when(s == 0)
    def _():
        e1_ref[:] = jnp.zeros_like(e1_ref)

    aug = jnp.concatenate(
        [fused, jnp.ones((tn, 1), jnp.float32), jnp.zeros((tn, _P1PAD), jnp.float32)],
        axis=1)
    e1_ref[:] += jax.lax.dot_general(aug, h, (((0,), (0,)), ((), ())),
                                     preferred_element_type=jnp.float32)


def _pass2_kernel(h_ref, e1_ref, dv_ref, w_ref, w1_ref, b1_ref, e2_ref):
    s = pl.program_id(1)
    half = w1_ref.shape[0]
    rows = half + 1 + _P1PAD
    e1t = e1_ref[0:half, :] + e1_ref[rows:rows + half, :]
    de = e1_ref[half:half + 1, :] + e1_ref[rows + half:rows + half + 1, :]
    sc = w_ref[:] / jnp.clip(de, _EPS, None)
    e1s = jnp.transpose(e1t * sc)
    h = h_ref[:]
    agg = jnp.dot(h, e1s, preferred_element_type=jnp.float32) / dv_ref[:]
    hid = jax.nn.relu(
        jnp.dot(agg, w1_ref[:], preferred_element_type=jnp.float32) + b1_ref[:])

    @pl.when(s == 0)
    def _():
        e2_ref[:] = jnp.zeros_like(e2_ref)

    e2_ref[:] += jax.lax.dot_general(hid, h, (((0,), (0,)), ((), ())),
                                     preferred_element_type=jnp.float32)


def _pass3_kernel(h_ref, e1_ref, e2_ref, dv_ref, w_ref, w2_ref, b2_ref,
                  wo_ref, bo_ref, out_ref):
    half = e1_ref.shape[0] // 2 - 1 - _P1PAD
    hid = w2_ref.shape[0]
    rows = half + 1 + _P1PAD
    de = e1_ref[half:half + 1, :] + e1_ref[rows + half:rows + half + 1, :]
    sc = w_ref[:] / jnp.clip(de, _EPS, None)
    e2t = e2_ref[0:hid, :] + e2_ref[hid:2 * hid, :]
    e2s = jnp.transpose(e2t * sc)
    agg = jnp.dot(h_ref[:], e2s, preferred_element_type=jnp.float32) / dv_ref[:]
    o = jax.nn.relu(
        jnp.dot(agg, w2_ref[:], preferred_element_type=jnp.float32) + b2_ref[:])
    out_ref[:] = jnp.dot(o, wo_ref[:], preferred_element_type=jnp.float32) + bo_ref[:]


def _pick_tile(n):
    for t in (200, 104, 100, 96, 80, 64, 56, 40, 32, 24, 16, 8):
        if n % (2 * t) == 0 and t % 8 == 0:
            return t
    return n


def kernel(x, z, incidence, edge_weights, Wpsi, bpsi, Wphi, bphi,
           Wg1, bg1, Wg2, bg2, W1, b1, W2, b2, Wo, bo):
    n, m = incidence.shape
    half = Wpsi.shape[1]
    hid = W1.shape[1]
    out_dim = Wo.shape[1]
    tn = _pick_tile(n)
    nsub = n // (2 * tn)
    grid = (2, nsub)
    p1rows = half + 1 + _P1PAD

    wcol = jnp.broadcast_to(edge_weights.reshape(m, 1), (m, 8))
    w2d = edge_weights.reshape(1, m)

    def row(b):
        return b.reshape(1, -1)

    def full(shape):
        return pl.BlockSpec(shape, lambda c, s: (0,) * len(shape))

    def tile(r, cdim):
        return pl.BlockSpec((r, cdim), lambda c, s: (c * nsub + s, 0))

    def chunk(r, cdim):
        return pl.BlockSpec((r, cdim), lambda c, s: (c, 0))

    f32 = jnp.float32
    params = pltpu.CompilerParams(dimension_semantics=("parallel", "arbitrary"))

    gate, e1, dv = pl.pallas_call(
        _pass1_kernel,
        grid=grid,
        in_specs=[tile(tn, x.shape[1]), tile(tn, z.shape[1]), tile(tn, m),
                  full((m, 8)),
                  full(Wpsi.shape), full((1, half)),
                  full(Wphi.shape), full((1, half)),
                  full(Wg1.shape), full((1, Wg1.shape[1])),
                  full(Wg2.shape), full((1, half))],
        out_specs=[tile(tn, half), chunk(p1rows, m), tile(tn, 1)],
        out_shape=[jax.ShapeDtypeStruct((n, half), f32),
                   jax.ShapeDtypeStruct((2 * p1rows, m), f32),
                   jax.ShapeDtypeStruct((n, 1), f32)],
        compiler_params=params,
    )(x, z, incidence, wcol, Wpsi, row(bpsi), Wphi, row(bphi),
      Wg1, row(bg1), Wg2, row(bg2))

    e2 = pl.pallas_call(
        _pass2_kernel,
        grid=grid,
        in_specs=[tile(tn, m), full((2 * p1rows, m)), tile(tn, 1),
                  full((1, m)), full(W1.shape), full((1, hid))],
        out_specs=chunk(hid, m),
        out_shape=jax.ShapeDtypeStruct((2 * hid, m), f32),
        compiler_params=params,
    )(incidence, e1, dv, w2d, W1, row(b1))

    logits = pl.pallas_call(
        _pass3_kernel,
        grid=grid,
        in_specs=[tile(tn, m), full((2 * p1rows, m)), full((2 * hid, m)),
                  tile(tn, 1),
                  full((1, m)), full(W2.shape), full((1, hid)),
                  full(Wo.shape), full((1, out_dim))],
        out_specs=tile(tn, out_dim),
        out_shape=jax.ShapeDtypeStruct((n, out_dim), f32),
        compiler_params=params,
    )(incidence, e1, e2, dv, w2d, W2, row(b2), Wo, row(bo))

    return (logits, gate)


# speedup vs baseline: 1.2059x; 1.2059x over previous
import jax
import jax.numpy as jnp
from jax.experimental import pallas as pl

_EPS = 1e-6
_P1PAD = 7
_CHUNK = 200


def _row_contract_acc(acc_ref, lhs, h_ref):
    tn = h_ref.shape[0]
    total = jnp.zeros_like(acc_ref)
    for k in range(0, tn, _CHUNK):
        total += jax.lax.dot_general(
            lhs[k:k + _CHUNK], h_ref[k:k + _CHUNK], (((0,), (0,)), ((), ())),
            preferred_element_type=jnp.float32)
    acc_ref[:] += total


def _pass1_kernel(x_ref, z_ref, h_ref,
                  wpsi_ref, bpsi_ref, wphi_ref, bphi_ref,
                  wg1_ref, bg1_ref, wg2_ref, bg2_ref,
                  gate_ref, e1_ref):
    i = pl.program_id(0)
    half = wpsi_ref.shape[1]
    tn = x_ref.shape[0]
    px = jnp.dot(x_ref[:], wpsi_ref[:], preferred_element_type=jnp.float32) + bpsi_ref[:]
    pz = jnp.dot(z_ref[:], wphi_ref[:], preferred_element_type=jnp.float32) + bphi_ref[:]
    g1 = jax.nn.relu(
        jnp.dot(px, wg1_ref[0:half, :], preferred_element_type=jnp.float32)
        + jnp.dot(pz, wg1_ref[half:, :], preferred_element_type=jnp.float32)
        + bg1_ref[:])
    gate = jax.nn.sigmoid(
        jnp.dot(g1, wg2_ref[:], preferred_element_type=jnp.float32) + bg2_ref[:])
    fused = gate * pz + (1.0 - gate) * px
    gate_ref[:] = gate

    @pl.when(i == 0)
    def _():
        e1_ref[:] = jnp.zeros_like(e1_ref)

    aug = jnp.concatenate(
        [fused, jnp.ones((tn, 1), jnp.float32), jnp.zeros((tn, _P1PAD), jnp.float32)],
        axis=1)
    _row_contract_acc(e1_ref, aug, h_ref)


def _agg_with_dv(h, et_scaled, w2d):
    k = et_scaled.shape[0]
    ea = jnp.transpose(jnp.concatenate([et_scaled, w2d], axis=0))
    res = jnp.dot(h, ea, preferred_element_type=jnp.float32)
    dv = jnp.clip(res[:, k:k + 1], _EPS, None)
    return res[:, 0:k] / dv


def _pass2_kernel(h_ref, e1_ref, w_ref, w1_ref, b1_ref, e2_ref):
    i = pl.program_id(0)
    half = w1_ref.shape[0]
    de = e1_ref[half:half + 1, :]
    sc = w_ref[:] / jnp.clip(de, _EPS, None)
    agg = _agg_with_dv(h_ref[:], e1_ref[0:half, :] * sc, w_ref[:])
    hid = jax.nn.relu(
        jnp.dot(agg, w1_ref[:], preferred_element_type=jnp.float32) + b1_ref[:])

    @pl.when(i == 0)
    def _():
        e2_ref[:] = jnp.zeros_like(e2_ref)

    _row_contract_acc(e2_ref, hid, h_ref)


def _pass3_kernel(h_ref, e1_ref, e2_ref, w_ref, w2_ref, b2_ref,
                  wo_ref, bo_ref, out_ref):
    hid = w2_ref.shape[0]
    half = e1_ref.shape[0] - 1 - _P1PAD
    de = e1_ref[half:half + 1, :]
    sc = w_ref[:] / jnp.clip(de, _EPS, None)
    agg = _agg_with_dv(h_ref[:], e2_ref[:] * sc, w_ref[:])
    o = jax.nn.relu(
        jnp.dot(agg, w2_ref[:], preferred_element_type=jnp.float32) + b2_ref[:])
    out_ref[:] = jnp.dot(o, wo_ref[:], preferred_element_type=jnp.float32) + bo_ref[:]


def _pick_tile(n):
    for t in (1000, 800, 600, 400, 200, 104, 100, 96, 80, 64, 56, 40, 32, 24, 16, 8):
        if n % t == 0 and t % 8 == 0:
            return t
    return n


def kernel(x, z, incidence, edge_weights, Wpsi, bpsi, Wphi, bphi,
           Wg1, bg1, Wg2, bg2, W1, b1, W2, b2, Wo, bo):
    n, m = incidence.shape
    half = Wpsi.shape[1]
    hid = W1.shape[1]
    out_dim = Wo.shape[1]
    tn = _pick_tile(n)
    grid = (n // tn,)
    p1rows = half + 1 + _P1PAD

    w2d = edge_weights.reshape(1, m)

    def row(b):
        return b.reshape(1, -1)

    def full(shape):
        return pl.BlockSpec(shape, lambda i: (0,) * len(shape))

    def tile(r, cdim):
        return pl.BlockSpec((r, cdim), lambda i: (i, 0))

    f32 = jnp.float32

    gate, e1 = pl.pallas_call(
        _pass1_kernel,
        grid=grid,
        in_specs=[tile(tn, x.shape[1]), tile(tn, z.shape[1]), tile(tn, m),
                  full(Wpsi.shape), full((1, half)),
                  full(Wphi.shape), full((1, half)),
                  full(Wg1.shape), full((1, Wg1.shape[1])),
                  full(Wg2.shape), full((1, half))],
        out_specs=[tile(tn, half), full((p1rows, m))],
        out_shape=[jax.ShapeDtypeStruct((n, half), f32),
                   jax.ShapeDtypeStruct((p1rows, m), f32)],
    )(x, z, incidence, Wpsi, row(bpsi), Wphi, row(bphi),
      Wg1, row(bg1), Wg2, row(bg2))

    e2 = pl.pallas_call(
        _pass2_kernel,
        grid=grid,
        in_specs=[tile(tn, m), full((p1rows, m)),
                  full((1, m)), full(W1.shape), full((1, hid))],
        out_specs=full((hid, m)),
        out_shape=jax.ShapeDtypeStruct((hid, m), f32),
    )(incidence, e1, w2d, W1, row(b1))

    logits = pl.pallas_call(
        _pass3_kernel,
        grid=grid,
        in_specs=[tile(tn, m), full((p1rows, m)), full((hid, m)),
                  full((1, m)), full(W2.shape), full((1, hid)),
                  full(Wo.shape), full((1, out_dim))],
        out_specs=tile(tn, out_dim),
        out_shape=jax.ShapeDtypeStruct((n, out_dim), f32),
    )(incidence, e1, e2, w2d, W2, row(b2), Wo, row(bo))

    return (logits, gate)
